# baseline (speedup 1.0000x reference)
import functools

import jax
import jax.numpy as jnp
from jax import lax
from jax.experimental import pallas as pl
from jax.experimental.pallas import tpu as pltpu
from jax.experimental.pallas import tpu_sc as plsc

NC = 2
NS = 16
NW = NC * NS
L = 16

ROWS_OUT = 127
N = 32768
S = N // L


def _scan_body(x_hbm, out_hbm, in_row, out_row):
    wid = lax.axis_index("s") * NC + lax.axis_index("c")
    base_idx = lax.iota(jnp.int32, L) * S

    for k in range((ROWS_OUT + NW - 1) // NW):
        row = wid + NW * k

        @pl.when(row < ROWS_OUT)
        def _():
            pltpu.sync_copy(x_hbm.at[row], in_row)

            def p1(i, acc):
                return acc + plsc.load_gather(in_row, [base_idx + i])

            seg = lax.fori_loop(0, S, p1, jnp.zeros((L,), jnp.float32))
            off = plsc.cumsum(seg) - seg

            def p2(i, run):
                v = plsc.load_gather(in_row, [base_idx + i])
                run = run + v
                plsc.store_scatter(out_row, [base_idx + i], run)
                return run

            lax.fori_loop(0, S, p2, off)
            pltpu.sync_copy(out_row, out_hbm.at[row])


@jax.jit
def _inclusive_scan(x):
    mesh = plsc.VectorSubcoreMesh(core_axis_name="c", subcore_axis_name="s")
    return pl.kernel(
        _scan_body,
        out_type=jax.ShapeDtypeStruct((ROWS_OUT, N), jnp.float32),
        mesh=mesh,
        scratch_types=[
            pltpu.VMEM((N,), jnp.float32),
            pltpu.VMEM((N,), jnp.float32),
        ],
        compiler_params=pltpu.CompilerParams(needs_layout_passes=False),
    )(x)


def kernel(x):
    incl = _inclusive_scan(x)
    zeros = jnp.zeros((ROWS_OUT, 1), x.dtype)
    return jnp.concatenate([zeros, incl], axis=1)


# speedup vs baseline: 1.4177x; 1.4177x over previous
import functools

import jax
import jax.numpy as jnp
from jax import lax
from jax.experimental import pallas as pl
from jax.experimental.pallas import tpu as pltpu
from jax.experimental.pallas import tpu_sc as plsc

NC = 2
NS = 16
NW = NC * NS
L = 16

ROWS_OUT = 127
N = 32768
S = N // L


U = 8


def _scan_body(x_hbm, out_hbm, in_row, out_row):
    wid = lax.axis_index("s") * NC + lax.axis_index("c")
    base_idx = lax.iota(jnp.int32, L) * S
    idx_u = [base_idx + u for u in range(U)]

    for k in range((ROWS_OUT + NW - 1) // NW):
        row = wid + NW * k

        @pl.when(row < ROWS_OUT)
        def _():
            pltpu.sync_copy(x_hbm.at[row], in_row)

            def p1(i, accs):
                g0 = i * U
                return tuple(
                    accs[u] + plsc.load_gather(in_row, [idx_u[u] + g0])
                    for u in range(U)
                )

            accs = lax.fori_loop(
                0, S // U, p1, tuple(jnp.zeros((L,), jnp.float32) for _ in range(U))
            )
            seg = accs[0]
            for u in range(1, U):
                seg = seg + accs[u]
            off = plsc.cumsum(seg) - seg

            def p2(i, run):
                g0 = i * U
                vs = [plsc.load_gather(in_row, [idx_u[u] + g0]) for u in range(U)]
                pref = [vs[0]]
                for u in range(1, U):
                    pref.append(pref[u - 1] + vs[u])
                plsc.store_scatter(out_row, [idx_u[0] + g0], run + vs[0])
                for u in range(1, U):
                    plsc.store_scatter(out_row, [idx_u[u] + g0], run + pref[u])
                return run + pref[U - 1]

            lax.fori_loop(0, S // U, p2, off)
            pltpu.sync_copy(out_row, out_hbm.at[row])


@jax.jit
def _inclusive_scan(x):
    mesh = plsc.VectorSubcoreMesh(core_axis_name="c", subcore_axis_name="s")
    return pl.kernel(
        _scan_body,
        out_type=jax.ShapeDtypeStruct((ROWS_OUT, N), jnp.float32),
        mesh=mesh,
        scratch_types=[
            pltpu.VMEM((N,), jnp.float32),
            pltpu.VMEM((N,), jnp.float32),
        ],
        compiler_params=pltpu.CompilerParams(needs_layout_passes=False),
    )(x)


def kernel(x):
    incl = _inclusive_scan(x)
    zeros = jnp.zeros((ROWS_OUT, 1), x.dtype)
    return jnp.concatenate([zeros, incl], axis=1)


# speedup vs baseline: 3.7424x; 2.6398x over previous
import functools

import jax
import jax.numpy as jnp
from jax import lax
from jax.experimental import pallas as pl
from jax.experimental.pallas import tpu as pltpu
from jax.experimental.pallas import tpu_sc as plsc

NC = 2
NS = 16
NW = NC * NS
L = 16

ROWS_OUT = 127
N = 32768
S = N // L
U = 8
M = S - L
MU = M // U


def _scan_body(x_hbm, out_hbm, in_row, out_row):
    wid = lax.axis_index("s") * NC + lax.axis_index("c")
    lane = lax.iota(jnp.int32, L)
    start_vec = lane * S
    end_vec = start_vec + S
    diag0 = start_vec + lane
    zero_f = jnp.zeros((L,), jnp.float32)

    for k in range((ROWS_OUT + NW - 1) // NW):
        row = wid + NW * k

        @pl.when(row < ROWS_OUT)
        def _():
            pltpu.sync_copy(x_hbm.at[row], in_row)

            def p1(i, accs):
                g0 = i * U
                return tuple(
                    accs[u] + plsc.load_gather(in_row, [diag0 + (g0 + u)])
                    for u in range(U)
                )

            accs = lax.fori_loop(0, MU, p1, tuple(zero_f for _ in range(U)))
            seg = accs[0]
            for u in range(1, U):
                seg = seg + accs[u]

            presum = zero_f
            for i in range(M, S):
                w = (diag0 + i) >= end_vec
                addr = jnp.where(w, diag0 + (i - S), diag0 + i)
                v = plsc.load_gather(in_row, [addr])
                seg = seg + v
                presum = presum + jnp.where(w, v, zero_f)

            off = plsc.cumsum(seg) - seg

            def p2(i, run):
                g0 = i * U
                vs = [
                    plsc.load_gather(in_row, [diag0 + (g0 + u)]) for u in range(U)
                ]
                pref = [vs[0]]
                for u in range(1, U):
                    pref.append(pref[u - 1] + vs[u])
                plsc.store_scatter(out_row, [diag0 + g0], run + vs[0])
                for u in range(1, U):
                    plsc.store_scatter(out_row, [diag0 + (g0 + u)], run + pref[u])
                return run + pref[U - 1]

            run = lax.fori_loop(0, MU, p2, off + presum)

            for i in range(M, S):
                w = (diag0 + i) >= end_vec
                wfirst = (diag0 + i) == end_vec
                addr = jnp.where(w, diag0 + (i - S), diag0 + i)
                run = jnp.where(wfirst, off, run)
                v = plsc.load_gather(in_row, [addr])
                run = run + v
                plsc.store_scatter(out_row, [addr], run)

            pltpu.sync_copy(out_row, out_hbm.at[row])


@jax.jit
def _inclusive_scan(x):
    mesh = plsc.VectorSubcoreMesh(core_axis_name="c", subcore_axis_name="s")
    return pl.kernel(
        _scan_body,
        out_type=jax.ShapeDtypeStruct((ROWS_OUT, N), jnp.float32),
        mesh=mesh,
        scratch_types=[
            pltpu.VMEM((N,), jnp.float32),
            pltpu.VMEM((N,), jnp.float32),
        ],
        compiler_params=pltpu.CompilerParams(needs_layout_passes=False),
    )(x)


def kernel(x):
    incl = _inclusive_scan(x)
    zeros = jnp.zeros((ROWS_OUT, 1), x.dtype)
    return jnp.concatenate([zeros, incl], axis=1)
